# baseline (speedup 1.0000x reference)
import functools

import jax
import jax.numpy as jnp
from jax.experimental import pallas as pl
from jax.experimental.pallas import tpu as pltpu

F = 256


def _lrelu(t):
    return jnp.where(t >= 0, t, 0.01 * t)


def _mm_bias_act_kernel(x_ref, w_ref, b_ref, o_ref, *, act):
    o_ref[...] = act(
        jnp.dot(x_ref[...], w_ref[...], preferred_element_type=jnp.float32)
        + b_ref[...])


def _mm_bias_act(x, w, b, act, block_rows):
    rows, k = x.shape
    m = w.shape[1]
    grid = rows // block_rows
    return pl.pallas_call(
        functools.partial(_mm_bias_act_kernel, act=act),
        grid=(grid,),
        in_specs=[
            pl.BlockSpec((block_rows, k), lambda i: (i, 0)),
            pl.BlockSpec((k, m), lambda i: (0, 0)),
            pl.BlockSpec((1, m), lambda i: (0, 0)),
        ],
        out_specs=pl.BlockSpec((block_rows, m), lambda i: (i, 0)),
        out_shape=jax.ShapeDtypeStruct((rows, m), jnp.float32),
    )(x, w, b)


def _edge_precompute_kernel(ea_ref, we_ref, be_ref, wall_ref, ball_ref, o_ref):
    ea = _lrelu(
        jnp.dot(ea_ref[...], we_ref[...], preferred_element_type=jnp.float32)
        + be_ref[...])
    o_ref[...] = (
        jnp.dot(ea, wall_ref[...], preferred_element_type=jnp.float32)
        + ball_ref[...])


def _edge_precompute(edge_attr_p, w_e, b_e, w_all, b_all, block_rows=640):
    e, k = edge_attr_p.shape
    m = w_all.shape[1]
    return pl.pallas_call(
        _edge_precompute_kernel,
        grid=(e // block_rows,),
        in_specs=[
            pl.BlockSpec((block_rows, k), lambda i: (i, 0)),
            pl.BlockSpec((k, F), lambda i: (0, 0)),
            pl.BlockSpec((1, F), lambda i: (0, 0)),
            pl.BlockSpec((F, m), lambda i: (0, 0)),
            pl.BlockSpec((1, m), lambda i: (0, 0)),
        ],
        out_specs=pl.BlockSpec((block_rows, m), lambda i: (i, 0)),
        out_shape=jax.ShapeDtypeStruct((e, m), jnp.float32),
    )(edge_attr_p, w_e, b_e, w_all, b_all)


def _msg_kernel(zf_ref, zs_ref, o_ref):
    zf = zf_ref[...]
    zs = zs_ref[...]
    sig = 1.0 / (1.0 + jnp.exp(-zf))
    sp = jnp.maximum(zs, 0.0) + jnp.log1p(jnp.exp(-jnp.abs(zs)))
    o_ref[...] = sig * sp


def _msg(zf, zs, block_rows=1280):
    e, f = zf.shape
    return pl.pallas_call(
        _msg_kernel,
        grid=(e // block_rows,),
        in_specs=[
            pl.BlockSpec((block_rows, f), lambda i: (i, 0)),
            pl.BlockSpec((block_rows, f), lambda i: (i, 0)),
        ],
        out_specs=pl.BlockSpec((block_rows, f), lambda i: (i, 0)),
        out_shape=jax.ShapeDtypeStruct((e, f), jnp.float32),
    )(zf, zs)


def _bn_stats_kernel(x_ref, o_ref):
    @pl.when(pl.program_id(0) == 0)
    def _():
        o_ref[...] = jnp.zeros_like(o_ref)

    x = x_ref[...]
    s1 = jnp.sum(x, axis=0, keepdims=True)
    s2 = jnp.sum(x * x, axis=0, keepdims=True)
    o_ref[...] += jnp.concatenate([s1, s2], axis=0)


def _bn_apply_kernel(stats_ref, agg_ref, h_ref, g_ref, b_ref, o_ref, *, n):
    s = stats_ref[...]
    mean = s[0:1, :] / n
    var = s[1:2, :] / n - mean * mean
    rstd = jax.lax.rsqrt(var + 1e-5)
    o_ref[...] = (agg_ref[...] - mean) * (rstd * g_ref[...]) + b_ref[...] \
        + h_ref[...]


def _bn_residual(agg, h, gamma, beta, block_rows=2000):
    n, f = agg.shape
    stats = pl.pallas_call(
        _bn_stats_kernel,
        grid=(n // block_rows,),
        in_specs=[pl.BlockSpec((block_rows, f), lambda i: (i, 0))],
        out_specs=pl.BlockSpec((2, f), lambda i: (0, 0)),
        out_shape=jax.ShapeDtypeStruct((2, f), jnp.float32),
    )(agg)
    return pl.pallas_call(
        functools.partial(_bn_apply_kernel, n=float(n)),
        grid=(n // block_rows,),
        in_specs=[
            pl.BlockSpec((2, f), lambda i: (0, 0)),
            pl.BlockSpec((block_rows, f), lambda i: (i, 0)),
            pl.BlockSpec((block_rows, f), lambda i: (i, 0)),
            pl.BlockSpec((1, f), lambda i: (0, 0)),
            pl.BlockSpec((1, f), lambda i: (0, 0)),
        ],
        out_specs=pl.BlockSpec((block_rows, f), lambda i: (i, 0)),
        out_shape=jax.ShapeDtypeStruct((n, f), jnp.float32),
    )(stats, agg, h, gamma.reshape(1, f), beta.reshape(1, f))


def _pool_kernel(h_ref, batch_ref, gw1_ref, gb1_ref, gw2_ref, gb2_ref,
                 nw1_ref, nb1_ref, nw2_ref, nb2_ref,
                 h1w_ref, h1b_ref, h2w_ref, h2b_ref, ow_ref, ob_ref, o_ref,
                 *, num_graphs):
    h = h_ref[...]
    g = (jnp.dot(jnp.maximum(
        jnp.dot(h, gw1_ref[...], preferred_element_type=jnp.float32)
        + gb1_ref[...], 0.0), gw2_ref[...],
        preferred_element_type=jnp.float32) + gb2_ref[...])
    g = g[:, 0:1]
    t = (jnp.dot(jnp.maximum(
        jnp.dot(h, nw1_ref[...], preferred_element_type=jnp.float32)
        + nb1_ref[...], 0.0), nw2_ref[...],
        preferred_element_type=jnp.float32) + nb2_ref[...])
    bcol = batch_ref[...]
    seg = jax.lax.broadcasted_iota(jnp.int32, (1, num_graphs), 1)
    msk = (bcol == seg)
    gmax = jnp.max(jnp.where(msk, g, -jnp.inf), axis=0, keepdims=True)
    gnode = jnp.sum(jnp.where(msk, gmax, 0.0), axis=1, keepdims=True)
    gexp = jnp.exp(g - gnode)
    mskf = msk.astype(jnp.float32)
    gsum = jax.lax.dot_general(mskf, gexp, (((0,), (0,)), ((), ())),
                               preferred_element_type=jnp.float32)
    st = jax.lax.dot_general(mskf, gexp * t, (((0,), (0,)), ((), ())),
                             preferred_element_type=jnp.float32)
    out = st / gsum
    out = _lrelu(jnp.dot(out, h1w_ref[...],
                         preferred_element_type=jnp.float32) + h1b_ref[...])
    out = _lrelu(jnp.dot(out, h2w_ref[...],
                         preferred_element_type=jnp.float32) + h2b_ref[...])
    o_ref[...] = (jnp.dot(out, ow_ref[...],
                          preferred_element_type=jnp.float32) + ob_ref[...])


def _pool(h, batch, params, num_graphs):
    n, f = h.shape
    gw2 = jnp.pad(params['gate_W2'], ((0, 0), (0, 127)))
    gb2 = jnp.pad(params['gate_b2'].reshape(1, 1), ((0, 0), (0, 127)))
    ow = jnp.pad(params['W_out'], ((0, 0), (0, 127)))
    ob = jnp.pad(params['b_out'].reshape(1, 1), ((0, 0), (0, 127)))
    out = pl.pallas_call(
        functools.partial(_pool_kernel, num_graphs=num_graphs),
        in_specs=[
            pl.BlockSpec((n, f), lambda: (0, 0)),
            pl.BlockSpec((n, 1), lambda: (0, 0)),
            pl.BlockSpec((f, f), lambda: (0, 0)),
            pl.BlockSpec((1, f), lambda: (0, 0)),
            pl.BlockSpec((f, 128), lambda: (0, 0)),
            pl.BlockSpec((1, 128), lambda: (0, 0)),
            pl.BlockSpec((f, f), lambda: (0, 0)),
            pl.BlockSpec((1, f), lambda: (0, 0)),
            pl.BlockSpec((f, f), lambda: (0, 0)),
            pl.BlockSpec((1, f), lambda: (0, 0)),
            pl.BlockSpec((f, f), lambda: (0, 0)),
            pl.BlockSpec((1, f), lambda: (0, 0)),
            pl.BlockSpec((f, f), lambda: (0, 0)),
            pl.BlockSpec((1, f), lambda: (0, 0)),
            pl.BlockSpec((f, 128), lambda: (0, 0)),
            pl.BlockSpec((1, 128), lambda: (0, 0)),
        ],
        out_specs=pl.BlockSpec((num_graphs, 128), lambda: (0, 0)),
        out_shape=jax.ShapeDtypeStruct((num_graphs, 128), jnp.float32),
    )(h, batch.reshape(n, 1).astype(jnp.int32),
      params['gate_W1'], params['gate_b1'].reshape(1, f), gw2, gb2,
      params['nn_W1'], params['nn_b1'].reshape(1, f),
      params['nn_W2'], params['nn_b2'].reshape(1, f),
      params['heads'][0]['W'], params['heads'][0]['b'].reshape(1, f),
      params['heads'][1]['W'], params['heads'][1]['b'].reshape(1, f),
      ow, ob)
    return out[:, 0]


def kernel(x, edge_attr, params, edge_index, batch):
    n, node_in = x.shape
    e, edge_in = edge_attr.shape
    num_graphs = 64
    num_layers = len(params['convs'])

    src = edge_index[0].astype(jnp.int32)
    dst = edge_index[1].astype(jnp.int32)

    kp = 256
    x_p = jnp.pad(x, ((0, 0), (0, kp - node_in)))
    wn_p = jnp.pad(params['W_node'], ((0, kp - node_in), (0, 0)))
    h = _mm_bias_act(x_p, wn_p, params['b_node'].reshape(1, F), _lrelu, 2000)

    kpe = 16
    ea_p = jnp.pad(edge_attr, ((0, 0), (0, kpe - edge_in)))
    we_p = jnp.pad(params['W_edge'], ((0, kpe - edge_in), (0, 0)))
    w_all = jnp.concatenate(
        [jnp.concatenate([p['Wf'][2 * F:], p['Ws'][2 * F:]], axis=1)
         for p in params['convs']], axis=1)
    b_all = jnp.concatenate(
        [jnp.concatenate([p['bf'], p['bs']]) for p in params['convs']])
    ea_proj = _edge_precompute(ea_p, we_p, params['b_edge'].reshape(1, F),
                               w_all, b_all.reshape(1, -1))

    ident = lambda t: t
    zero_bias = jnp.zeros((1, 4 * F), jnp.float32)
    for li, p in enumerate(params['convs']):
        w_cat = jnp.concatenate(
            [p['Wf'][:F], p['Ws'][:F], p['Wf'][F:2 * F], p['Ws'][F:2 * F]],
            axis=1)
        proj = _mm_bias_act(h, w_cat, zero_bias, ident, 2000)
        gd = jnp.take(proj[:, :2 * F], dst, axis=0)
        gs = jnp.take(proj[:, 2 * F:], src, axis=0)
        zf = gd[:, :F] + gs[:, :F] + ea_proj[:, li * 2 * F: li * 2 * F + F]
        zs = gd[:, F:] + gs[:, F:] + ea_proj[:, li * 2 * F + F:(li + 1) * 2 * F]
        m = _msg(zf, zs)
        agg = jax.ops.segment_sum(m, dst, num_segments=n)
        h = _bn_residual(agg, h, p['gamma'], p['beta'])

    return _pool(h, batch, params, num_graphs)
